# baseline (speedup 1.0000x reference)
import jax
import jax.numpy as jnp
from jax.experimental import pallas as pl
from jax.experimental.pallas import tpu as pltpu

TOPK = 8
LANES = 128
NSTREAMS = 4


def _router_body(x0, x1, x2, x3, gw_ref, gb_ref, eb_ref, probs_out, idx_out,
                 xm_scr):
    b = pl.program_id(0)
    nb = pl.num_programs(0)
    spatial = x0.shape[2]
    nsub = spatial // LANES

    for k, xk in enumerate((x0, x1, x2, x3)):
        xs = xk[0]
        acc = xs[:, 0:LANES]
        for j in range(1, nsub):
            acc = acc + xs[:, j * LANES:(j + 1) * LANES]
        xm_scr[NSTREAMS * b + k] = acc

    @pl.when(b == nb - 1)
    def _finish():
        xm = jnp.sum(xm_scr[...], axis=2) * (1.0 / spatial)
        nrows, nexp = xm.shape[0], gw_ref.shape[0]
        logits = jax.lax.dot_general(
            xm, gw_ref[...], (((1,), (1,)), ((), ())),
            preferred_element_type=jnp.float32)
        logits = logits + gb_ref[...]
        logits = jnp.clip(logits, -10.0, 10.0)
        lb = logits + eb_ref[...]
        m = jnp.max(lb, axis=1, keepdims=True)
        e = jnp.exp(lb - m)
        p = e / jnp.sum(e, axis=1, keepdims=True)
        p = jnp.clip(p, 1e-06, 1.0)
        iota = jax.lax.broadcasted_iota(jnp.int32, (nrows, nexp), 1)
        vals, idxs = [], []
        for _ in range(TOPK):
            mk = jnp.max(p, axis=1, keepdims=True)
            ik = jnp.min(jnp.where(p == mk, iota, nexp), axis=1, keepdims=True)
            vals.append(mk)
            idxs.append(ik)
            p = jnp.where(iota == ik, -jnp.inf, p)
        tv = jnp.concatenate(vals, axis=1)
        ti = jnp.concatenate(idxs, axis=1)
        tv = tv / (jnp.sum(tv, axis=1, keepdims=True) + 1e-08)
        probs_out[...] = tv
        idx_out[...] = ti


def kernel(x, gate_w, gate_b, expert_bias):
    B, C, H, W = x.shape
    E = gate_w.shape[0]
    S = H * W
    xr = x.reshape(B, C, S)
    gb = gate_b.reshape(1, E)
    eb = expert_bias.reshape(1, E)

    def xspec(k):
        return pl.BlockSpec((1, C, S), lambda b: (NSTREAMS * b + k, 0, 0))

    probs, idx = pl.pallas_call(
        _router_body,
        grid=(B // NSTREAMS,),
        in_specs=[
            xspec(0), xspec(1), xspec(2), xspec(3),
            pl.BlockSpec((E, C), lambda b: (0, 0)),
            pl.BlockSpec((1, E), lambda b: (0, 0)),
            pl.BlockSpec((1, E), lambda b: (0, 0)),
        ],
        out_specs=[
            pl.BlockSpec((B, TOPK), lambda b: (0, 0)),
            pl.BlockSpec((B, TOPK), lambda b: (0, 0)),
        ],
        out_shape=[
            jax.ShapeDtypeStruct((B, TOPK), jnp.float32),
            jax.ShapeDtypeStruct((B, TOPK), jnp.int32),
        ],
        scratch_shapes=[pltpu.VMEM((B, C, LANES), jnp.float32)],
    )(xr, xr, xr, xr, gate_w, gb, eb)

    loss = jnp.zeros((), dtype=jnp.float32)
    return (probs, idx, loss)
